# baseline (speedup 1.0000x reference)
import functools

import jax
import jax.numpy as jnp
from jax import lax
from jax.experimental import pallas as pl
from jax.experimental.pallas import tpu as pltpu
from jax.experimental.pallas import tpu_sc as plsc

_EPS = 1e-12
_NC = 2
_NS = 16
_NW = _NC * _NS
_G = 128
_CH = 64


def _pass_body(bank_ref, ir_ref, qr_ref, packed_ref):
    nb, K2, D2 = bank_ref.shape
    D = D2 // 2
    K = 2 * K2
    itm_all = ir_ref[:, 0, :]
    qry_all = qr_ref[:, 0, :]

    inv_i = 1.0 / jnp.maximum(jnp.sqrt(jnp.sum(itm_all * itm_all, axis=1, keepdims=True)), _EPS)
    inv_q = 1.0 / jnp.maximum(jnp.sqrt(jnp.sum(qry_all * qry_all, axis=1, keepdims=True)), _EPS)

    z = jnp.zeros_like(itm_all)
    V = jnp.concatenate([
        jnp.concatenate([itm_all, z], axis=1),
        jnp.concatenate([z, itm_all], axis=1),
        jnp.concatenate([qry_all, z], axis=1),
        jnp.concatenate([z, qry_all], axis=1),
    ], axis=0)
    lane2 = jax.lax.broadcasted_iota(jnp.int32, (8, D2), 1)
    sub2 = jax.lax.broadcasted_iota(jnp.int32, (8, D2), 0)
    ones2 = jnp.where(jnp.logical_and(sub2 == 0, lane2 < D), 1.0, 0.0) \
        + jnp.where(jnp.logical_and(sub2 == 1, lane2 >= D), 1.0, 0.0)

    die, dio, dqe, dqo, nse, nso = [], [], [], [], [], []
    for s in range(nb):
        bank2 = bank_ref[s]
        MX = jax.lax.dot_general(V, bank2, (((1,), (1,)), ((), ())),
                                 preferred_element_type=jnp.float32)
        NO = jax.lax.dot_general(ones2, bank2 * bank2, (((1,), (1,)), ((), ())),
                                 preferred_element_type=jnp.float32)
        die.append(MX[s:s + 1, :])
        dio.append(MX[nb + s:nb + s + 1, :])
        dqe.append(MX[2 * nb + s:2 * nb + s + 1, :])
        dqo.append(MX[3 * nb + s:3 * nb + s + 1, :])
        nse.append(NO[0:1, :])
        nso.append(NO[1:2, :])

    d_i_e = jnp.concatenate(die, axis=0)
    d_i_o = jnp.concatenate(dio, axis=0)
    d_q_e = jnp.concatenate(dqe, axis=0)
    d_q_o = jnp.concatenate(dqo, axis=0)
    nsq_e = jnp.concatenate(nse, axis=0)
    nsq_o = jnp.concatenate(nso, axis=0)

    inv_be = 1.0 / jnp.maximum(jnp.sqrt(nsq_e), _EPS)
    inv_bo = 1.0 / jnp.maximum(jnp.sqrt(nsq_o), _EPS)
    s_i_e = d_i_e * inv_be * inv_i
    s_i_o = d_i_o * inv_bo * inv_i
    s_q_e = d_q_e * inv_be * inv_q
    s_q_o = d_q_o * inv_bo * inv_q

    e_e = jnp.exp(s_q_e)
    e_o = jnp.exp(s_q_o)
    S = (jnp.sum(e_e, axis=1, keepdims=True)
         + jnp.sum(e_o, axis=1, keepdims=True))

    m = jnp.maximum(jnp.max(s_i_e, axis=1, keepdims=True),
                    jnp.max(s_i_o, axis=1, keepdims=True))
    jio = jax.lax.broadcasted_iota(jnp.int32, (nb, K2), 1)
    cand_e = jnp.where(s_i_e >= m, 2 * jio, K)
    cand_o = jnp.where(s_i_o >= m, 2 * jio + 1, K)
    idx = jnp.minimum(jnp.min(cand_e, axis=1, keepdims=True),
                      jnp.min(cand_o, axis=1, keepdims=True))
    oh_e = (2 * jio == idx).astype(jnp.float32)
    oh_o = (2 * jio + 1 == idx).astype(jnp.float32)
    sq_best = jnp.sum(oh_e * s_q_e + oh_o * s_q_o, axis=1, keepdims=True)
    sq_0 = s_q_e[:, 0:1]

    EO = jnp.concatenate([e_e, e_o, oh_e, oh_o], axis=0)
    r_rows, rb_rows, r0_rows = [], [], []
    for s in range(nb):
        bank2 = bank_ref[s]
        R4 = jax.lax.dot_general(EO, bank2, (((1,), (0,)), ((), ())),
                                 preferred_element_type=jnp.float32)
        r_rows.append(R4[s:s + 1, 0:D] + R4[nb + s:nb + s + 1, D:2 * D])
        rb_rows.append(R4[2 * nb + s:2 * nb + s + 1, 0:D]
                       + R4[3 * nb + s:3 * nb + s + 1, D:2 * D])
        r0_rows.append(bank2[0:1, 0:D])

    R = jnp.concatenate(r_rows, axis=0)
    row_best = jnp.concatenate(rb_rows, axis=0)
    row0 = jnp.concatenate(r0_rows, axis=0)

    e_best = jnp.exp(sq_best)
    e_0 = jnp.exp(sq_0)
    new_c = 0.5 * row_best + 0.5 * itm_all
    inv_nc = 1.0 / jnp.maximum(jnp.sqrt(jnp.sum(new_c * new_c, axis=1, keepdims=True)), _EPS)
    e_new_c = jnp.exp(jnp.sum(new_c * qry_all, axis=1, keepdims=True) * inv_nc * inv_q)
    e_new_o = jnp.exp(jnp.sum(itm_all * qry_all, axis=1, keepdims=True) * inv_i * inv_q)

    A_c = e_new_c * new_c - e_best * row_best
    A_o = e_new_o * itm_all - e_0 * row0
    dS_c = e_new_c - e_best
    dS_o = e_new_o - e_0

    dlane = jax.lax.broadcasted_iota(jnp.int32, (nb, D), 1)
    scal = (jnp.where(dlane == 0, S, 0.0)
            + jnp.where(dlane == 1, dS_c, 0.0)
            + jnp.where(dlane == 2, dS_o, 0.0)
            + jnp.where(dlane == 3, m, 0.0))

    packed_ref[:, 0, :] = R
    packed_ref[:, 1, :] = A_c
    packed_ref[:, 2, :] = A_o
    packed_ref[:, 3, :] = scal


def _rsqrt16(x):
    xi = plsc.bitcast(x, jnp.int32)
    y = plsc.bitcast(jnp.int32(0x5F3759DF) - (xi >> 1), jnp.float32)
    for _ in range(3):
        y = y * (1.5 - 0.5 * x * y * y)
    return y


def _sc_pass(bank2, item, query, B, K, D, start):
    K2 = K // 2
    D2 = 2 * D
    spw = _G // _NW
    chp = _CH // 2
    nchunks = K2 // chp
    ngroups = chp // 16
    mesh = plsc.VectorSubcoreMesh(core_axis_name="c", subcore_axis_name="s")

    @functools.partial(
        pl.kernel,
        out_type=jax.ShapeDtypeStruct((_G, 4, D), jnp.float32),
        mesh=mesh,
        compiler_params=pltpu.CompilerParams(needs_layout_passes=False,
                                             use_tc_tiling_on_sc=True),
        scratch_types=[
            pltpu.VMEM((_CH // 2, 2 * D), jnp.float32),
            pltpu.VMEM((1, 2 * D), jnp.float32),
            pltpu.VMEM((1, 2 * D), jnp.float32),
            pltpu.VMEM((D,), jnp.float32),
            pltpu.VMEM((D,), jnp.float32),
            pltpu.VMEM((4, D), jnp.float32),
            pltpu.SemaphoreType.DMA,
        ],
    )
    def sc_kernel(bank_hbm, item_hbm, query_hbm, out_hbm,
                  chunk_v, rb_v, r0_v, it_v, q_v, out_v, sem):
        w = lax.axis_index("s") * _NC + lax.axis_index("c")
        i16 = lax.broadcasted_iota(jnp.int32, (16,), 0)
        neg = jnp.full((16,), -3.0e38, jnp.float32)
        zero = jnp.zeros((16,), jnp.float32)
        zeroi = jnp.zeros((16,), jnp.int32)

        def stream_body(j, _):
            g = w * spw + j
            b = start + g
            pltpu.sync_copy(item_hbm.at[b], it_v)
            pltpu.sync_copy(query_hbm.at[b], q_v)
            it_vecs = [it_v[pl.ds(16 * t, 16)] for t in range(D // 16)]
            q_vecs = [q_v[pl.ds(16 * t, 16)] for t in range(D // 16)]

            def vnorm(vecs):
                ss = zero
                for v in vecs:
                    ss = ss + v * v
                return _rsqrt16(jnp.maximum(jnp.full((16,), jnp.sum(ss)),
                                            _EPS * _EPS))

            inv_i = vnorm(it_vecs)
            inv_q = vnorm(q_vecs)

            def chunk_body(c, carry):
                m_run, bidx, s_run, r0a, r1a, r2a, r3a = carry
                pltpu.sync_copy(bank_hbm.at[b, pl.ds(c * chp, chp)], chunk_v)
                e_groups = []
                for gi in range(ngroups):
                    rowv = i16 + gi * 16
                    acc = [zero] * 6
                    for d in range(D2):
                        colv = jnp.full((16,), d, jnp.int32)
                        v = plsc.load_gather(chunk_v, [rowv, colv])
                        dd = d % D
                        o = 0 if d < D else 3
                        acc[o] = acc[o] + v * it_vecs[dd // 16][dd % 16]
                        acc[o + 1] = acc[o + 1] + v * q_vecs[dd // 16][dd % 16]
                        acc[o + 2] = acc[o + 2] + v * v
                    pair_glob = rowv + c * chp
                    epair = []
                    for half in range(2):
                        a_i, a_q, a_n = acc[3 * half:3 * half + 3]
                        inv_b = _rsqrt16(jnp.maximum(a_n, _EPS * _EPS))
                        s_i = a_i * inv_b * inv_i
                        s_q = a_q * inv_b * inv_q
                        e = jnp.exp(s_q)
                        s_run = s_run + e
                        epair.append(e)
                        rows_glob = 2 * pair_glob + half
                        upd = s_i > m_run
                        m_run = jnp.where(upd, s_i, m_run)
                        bidx = jnp.where(upd, rows_glob, bidx)
                    e_groups.append(epair)
                for gi in range(ngroups):
                    ev_e, ev_o = e_groups[gi]
                    for rr in range(16):
                        r = gi * 16 + rr
                        ee = ev_e[rr]
                        eo = ev_o[rr]
                        r0a = (r0a + chunk_v[r, pl.ds(0, 16)] * ee
                               + chunk_v[r, pl.ds(64, 16)] * eo)
                        r1a = (r1a + chunk_v[r, pl.ds(16, 16)] * ee
                               + chunk_v[r, pl.ds(80, 16)] * eo)
                        r2a = (r2a + chunk_v[r, pl.ds(32, 16)] * ee
                               + chunk_v[r, pl.ds(96, 16)] * eo)
                        r3a = (r3a + chunk_v[r, pl.ds(48, 16)] * ee
                               + chunk_v[r, pl.ds(112, 16)] * eo)
                return (m_run, bidx, s_run, r0a, r1a, r2a, r3a)

            init = (neg, jnp.full((16,), K, jnp.int32), zero,
                    zero, zero, zero, zero)
            m_run, bidx, s_run, r0a, r1a, r2a, r3a = lax.fori_loop(
                0, nchunks, chunk_body, init)

            m_fin = jnp.max(m_run)
            cand = jnp.where(m_run >= m_fin, bidx, K)
            idx_fin = jnp.min(cand)
            s_fin = jnp.sum(s_run)

            pidx = idx_fin // 2
            parity = idx_fin - 2 * pidx
            pltpu.sync_copy(bank_hbm.at[b, pl.ds(pidx, 1)], rb_v)
            pltpu.sync_copy(bank_hbm.at[b, pl.ds(0, 1)], r0_v)

            out_v[0, pl.ds(0, 16)] = r0a
            out_v[0, pl.ds(16, 16)] = r1a
            out_v[0, pl.ds(32, 16)] = r2a
            out_v[0, pl.ds(48, 16)] = r3a
            par = zeroi + parity
            for t in range(D // 16):
                he = rb_v[0, pl.ds(16 * t, 16)]
                ho = rb_v[0, pl.ds(D + 16 * t, 16)]
                out_v[1, pl.ds(16 * t, 16)] = jnp.where(par == 0, he, ho)
                out_v[2, pl.ds(16 * t, 16)] = r0_v[0, pl.ds(16 * t, 16)]
            sc0 = (jnp.where(i16 == 0, jnp.full((16,), s_fin), 0.0)
                   + jnp.where(i16 == 3, jnp.full((16,), m_fin), 0.0))
            out_v[3, pl.ds(0, 16)] = sc0
            out_v[3, pl.ds(16, 16)] = zero
            out_v[3, pl.ds(32, 16)] = zero
            out_v[3, pl.ds(48, 16)] = zero
            pltpu.sync_copy(out_v, out_hbm.at[g])
            return 0

        lax.fori_loop(0, spw, stream_body, 0)

    return sc_kernel(bank2, item, query)


def _finalize_body(packed_ref, qr_ref, ir_ref, out_ref):
    pk = packed_ref[...]
    B = pk.shape[0]
    BT = B - _G
    cond = (jnp.sum(pk[:, 3, 3:4]) * (1.0 / B)) >= 0.5

    R_a = pk[0:BT, 0, :]
    S_a = pk[0:BT, 3, 0:1]
    S_fin_a = S_a + jnp.where(cond, pk[0:BT, 3, 1:2], pk[0:BT, 3, 2:3])
    R_fin_a = R_a + jnp.where(cond, pk[0:BT, 1, :], pk[0:BT, 2, :])
    out_ref[0:BT, :] = R_fin_a / S_fin_a

    R_b = pk[BT:, 0, :]
    rb = pk[BT:, 1, :]
    r0 = pk[BT:, 2, :]
    S_b = pk[BT:, 3, 0:1]
    qry = qr_ref[BT:, :]
    itm = ir_ref[BT:, :]

    def inv_norm(x):
        return 1.0 / jnp.maximum(jnp.sqrt(jnp.sum(x * x, axis=1, keepdims=True)), _EPS)

    def rdot(a, bv):
        return jnp.sum(a * bv, axis=1, keepdims=True)

    inv_q = inv_norm(qry)
    inv_i = inv_norm(itm)
    e_best = jnp.exp(rdot(qry, rb) * inv_norm(rb) * inv_q)
    e_0 = jnp.exp(rdot(qry, r0) * inv_norm(r0) * inv_q)
    new_c = 0.5 * rb + 0.5 * itm
    e_new_c = jnp.exp(rdot(qry, new_c) * inv_norm(new_c) * inv_q)
    e_new_o = jnp.exp(rdot(qry, itm) * inv_i * inv_q)

    S_fin_b = S_b + jnp.where(cond, e_new_c - e_best, e_new_o - e_0)
    R_fin_b = R_b + jnp.where(cond,
                              e_new_c * new_c - e_best * rb,
                              e_new_o * itm - e_0 * r0)
    out_ref[BT:, :] = R_fin_b / S_fin_b


def kernel(query, item, memory_bank):
    B, K, D = memory_bank.shape
    BT = B - _G
    q3 = query.reshape(B, 1, D)
    i3 = item.reshape(B, 1, D)
    bank2 = memory_bank.reshape(B, K // 2, 2 * D)

    BPB = 8
    packed_tc = pl.pallas_call(
        _pass_body,
        grid=(BT // BPB,),
        in_specs=[
            pl.BlockSpec((BPB, K // 2, 2 * D), lambda b: (b, 0, 0)),
            pl.BlockSpec((BPB, 1, D), lambda b: (b, 0, 0)),
            pl.BlockSpec((BPB, 1, D), lambda b: (b, 0, 0)),
        ],
        out_specs=pl.BlockSpec((BPB, 4, D), lambda b: (b, 0, 0)),
        out_shape=jax.ShapeDtypeStruct((BT, 4, D), jnp.float32),
    )(bank2, i3, q3)

    packed_sc = _sc_pass(bank2, item, query, B, K, D, BT)
    packed = jnp.concatenate([packed_tc, packed_sc], axis=0)

    retrieved = pl.pallas_call(
        _finalize_body,
        out_shape=jax.ShapeDtypeStruct((B, D), jnp.float32),
    )(packed, query, item)
    return retrieved


# speedup vs baseline: 1.5330x; 1.5330x over previous
import functools

import jax
import jax.numpy as jnp
from jax import lax
from jax.experimental import pallas as pl
from jax.experimental.pallas import tpu as pltpu
from jax.experimental.pallas import tpu_sc as plsc

_EPS = 1e-12
_NC = 2
_NS = 16
_NW = _NC * _NS
_G = 32
_CH = 64


def _pass_body(bank_ref, ir_ref, qr_ref, packed_ref):
    nb, K2, D2 = bank_ref.shape
    D = D2 // 2
    K = 2 * K2
    itm_all = ir_ref[:, 0, :]
    qry_all = qr_ref[:, 0, :]

    inv_i = 1.0 / jnp.maximum(jnp.sqrt(jnp.sum(itm_all * itm_all, axis=1, keepdims=True)), _EPS)
    inv_q = 1.0 / jnp.maximum(jnp.sqrt(jnp.sum(qry_all * qry_all, axis=1, keepdims=True)), _EPS)

    z = jnp.zeros_like(itm_all)
    V = jnp.concatenate([
        jnp.concatenate([itm_all, z], axis=1),
        jnp.concatenate([z, itm_all], axis=1),
        jnp.concatenate([qry_all, z], axis=1),
        jnp.concatenate([z, qry_all], axis=1),
    ], axis=0)
    lane2 = jax.lax.broadcasted_iota(jnp.int32, (8, D2), 1)
    sub2 = jax.lax.broadcasted_iota(jnp.int32, (8, D2), 0)
    ones2 = jnp.where(jnp.logical_and(sub2 == 0, lane2 < D), 1.0, 0.0) \
        + jnp.where(jnp.logical_and(sub2 == 1, lane2 >= D), 1.0, 0.0)

    die, dio, dqe, dqo, nse, nso = [], [], [], [], [], []
    for s in range(nb):
        bank2 = bank_ref[s]
        MX = jax.lax.dot_general(V, bank2, (((1,), (1,)), ((), ())),
                                 preferred_element_type=jnp.float32)
        NO = jax.lax.dot_general(ones2, bank2 * bank2, (((1,), (1,)), ((), ())),
                                 preferred_element_type=jnp.float32)
        die.append(MX[s:s + 1, :])
        dio.append(MX[nb + s:nb + s + 1, :])
        dqe.append(MX[2 * nb + s:2 * nb + s + 1, :])
        dqo.append(MX[3 * nb + s:3 * nb + s + 1, :])
        nse.append(NO[0:1, :])
        nso.append(NO[1:2, :])

    d_i_e = jnp.concatenate(die, axis=0)
    d_i_o = jnp.concatenate(dio, axis=0)
    d_q_e = jnp.concatenate(dqe, axis=0)
    d_q_o = jnp.concatenate(dqo, axis=0)
    nsq_e = jnp.concatenate(nse, axis=0)
    nsq_o = jnp.concatenate(nso, axis=0)

    inv_be = 1.0 / jnp.maximum(jnp.sqrt(nsq_e), _EPS)
    inv_bo = 1.0 / jnp.maximum(jnp.sqrt(nsq_o), _EPS)
    s_i_e = d_i_e * inv_be * inv_i
    s_i_o = d_i_o * inv_bo * inv_i
    s_q_e = d_q_e * inv_be * inv_q
    s_q_o = d_q_o * inv_bo * inv_q

    e_e = jnp.exp(s_q_e)
    e_o = jnp.exp(s_q_o)
    S = (jnp.sum(e_e, axis=1, keepdims=True)
         + jnp.sum(e_o, axis=1, keepdims=True))

    m = jnp.maximum(jnp.max(s_i_e, axis=1, keepdims=True),
                    jnp.max(s_i_o, axis=1, keepdims=True))
    jio = jax.lax.broadcasted_iota(jnp.int32, (nb, K2), 1)
    cand_e = jnp.where(s_i_e >= m, 2 * jio, K)
    cand_o = jnp.where(s_i_o >= m, 2 * jio + 1, K)
    idx = jnp.minimum(jnp.min(cand_e, axis=1, keepdims=True),
                      jnp.min(cand_o, axis=1, keepdims=True))
    oh_e = (2 * jio == idx).astype(jnp.float32)
    oh_o = (2 * jio + 1 == idx).astype(jnp.float32)
    sq_best = jnp.sum(oh_e * s_q_e + oh_o * s_q_o, axis=1, keepdims=True)
    sq_0 = s_q_e[:, 0:1]

    EO = jnp.concatenate([e_e, e_o, oh_e, oh_o], axis=0)
    r_rows, rb_rows, r0_rows = [], [], []
    for s in range(nb):
        bank2 = bank_ref[s]
        R4 = jax.lax.dot_general(EO, bank2, (((1,), (0,)), ((), ())),
                                 preferred_element_type=jnp.float32)
        r_rows.append(R4[s:s + 1, 0:D] + R4[nb + s:nb + s + 1, D:2 * D])
        rb_rows.append(R4[2 * nb + s:2 * nb + s + 1, 0:D]
                       + R4[3 * nb + s:3 * nb + s + 1, D:2 * D])
        r0_rows.append(bank2[0:1, 0:D])

    R = jnp.concatenate(r_rows, axis=0)
    row_best = jnp.concatenate(rb_rows, axis=0)
    row0 = jnp.concatenate(r0_rows, axis=0)

    e_best = jnp.exp(sq_best)
    e_0 = jnp.exp(sq_0)
    new_c = 0.5 * row_best + 0.5 * itm_all
    inv_nc = 1.0 / jnp.maximum(jnp.sqrt(jnp.sum(new_c * new_c, axis=1, keepdims=True)), _EPS)
    e_new_c = jnp.exp(jnp.sum(new_c * qry_all, axis=1, keepdims=True) * inv_nc * inv_q)
    e_new_o = jnp.exp(jnp.sum(itm_all * qry_all, axis=1, keepdims=True) * inv_i * inv_q)

    A_c = e_new_c * new_c - e_best * row_best
    A_o = e_new_o * itm_all - e_0 * row0
    dS_c = e_new_c - e_best
    dS_o = e_new_o - e_0

    dlane = jax.lax.broadcasted_iota(jnp.int32, (nb, D), 1)
    scal = (jnp.where(dlane == 0, S, 0.0)
            + jnp.where(dlane == 1, dS_c, 0.0)
            + jnp.where(dlane == 2, dS_o, 0.0)
            + jnp.where(dlane == 3, m, 0.0))

    packed_ref[:, 0, :] = R
    packed_ref[:, 1, :] = A_c
    packed_ref[:, 2, :] = A_o
    packed_ref[:, 3, :] = scal


def _rsqrt16(x):
    xi = plsc.bitcast(x, jnp.int32)
    y = plsc.bitcast(jnp.int32(0x5F3759DF) - (xi >> 1), jnp.float32)
    for _ in range(3):
        y = y * (1.5 - 0.5 * x * y * y)
    return y


def _sc_pass(bank2, item, query, B, K, D, start):
    K2 = K // 2
    D2 = 2 * D
    spw = _G // _NW
    chp = _CH // 2
    nchunks = K2 // chp
    ngroups = chp // 16
    mesh = plsc.VectorSubcoreMesh(core_axis_name="c", subcore_axis_name="s")

    @functools.partial(
        pl.kernel,
        out_type=jax.ShapeDtypeStruct((_G, 4, D), jnp.float32),
        mesh=mesh,
        compiler_params=pltpu.CompilerParams(needs_layout_passes=False,
                                             use_tc_tiling_on_sc=True),
        scratch_types=[
            pltpu.VMEM((_CH // 2, 2 * D), jnp.float32),
            pltpu.VMEM((1, 2 * D), jnp.float32),
            pltpu.VMEM((1, 2 * D), jnp.float32),
            pltpu.VMEM((D,), jnp.float32),
            pltpu.VMEM((D,), jnp.float32),
            pltpu.VMEM((4, D), jnp.float32),
            pltpu.SemaphoreType.DMA,
        ],
    )
    def sc_kernel(bank_hbm, item_hbm, query_hbm, out_hbm,
                  chunk_v, rb_v, r0_v, it_v, q_v, out_v, sem):
        w = lax.axis_index("s") * _NC + lax.axis_index("c")
        i16 = lax.broadcasted_iota(jnp.int32, (16,), 0)
        neg = jnp.full((16,), -3.0e38, jnp.float32)
        zero = jnp.zeros((16,), jnp.float32)
        zeroi = jnp.zeros((16,), jnp.int32)

        def stream_body(j, _):
            g = w * spw + j
            b = start + g
            pltpu.sync_copy(item_hbm.at[b], it_v)
            pltpu.sync_copy(query_hbm.at[b], q_v)
            it_vecs = [it_v[pl.ds(16 * t, 16)] for t in range(D // 16)]
            q_vecs = [q_v[pl.ds(16 * t, 16)] for t in range(D // 16)]

            def vnorm(vecs):
                ss = zero
                for v in vecs:
                    ss = ss + v * v
                return _rsqrt16(jnp.maximum(jnp.full((16,), jnp.sum(ss)),
                                            _EPS * _EPS))

            inv_i = vnorm(it_vecs)
            inv_q = vnorm(q_vecs)

            def chunk_body(c, carry):
                m_run, bidx, s_run, r0a, r1a, r2a, r3a = carry
                pltpu.sync_copy(bank_hbm.at[b, pl.ds(c * chp, chp)], chunk_v)
                e_groups = []
                for gi in range(ngroups):
                    rowv = i16 + gi * 16
                    acc = [zero] * 6
                    for d in range(D2):
                        colv = jnp.full((16,), d, jnp.int32)
                        v = plsc.load_gather(chunk_v, [rowv, colv])
                        dd = d % D
                        o = 0 if d < D else 3
                        acc[o] = acc[o] + v * it_vecs[dd // 16][dd % 16]
                        acc[o + 1] = acc[o + 1] + v * q_vecs[dd // 16][dd % 16]
                        acc[o + 2] = acc[o + 2] + v * v
                    pair_glob = rowv + c * chp
                    epair = []
                    for half in range(2):
                        a_i, a_q, a_n = acc[3 * half:3 * half + 3]
                        inv_b = _rsqrt16(jnp.maximum(a_n, _EPS * _EPS))
                        s_i = a_i * inv_b * inv_i
                        s_q = a_q * inv_b * inv_q
                        e = jnp.exp(s_q)
                        s_run = s_run + e
                        epair.append(e)
                        rows_glob = 2 * pair_glob + half
                        upd = s_i > m_run
                        m_run = jnp.where(upd, s_i, m_run)
                        bidx = jnp.where(upd, rows_glob, bidx)
                    e_groups.append(epair)
                for gi in range(ngroups):
                    ev_e, ev_o = e_groups[gi]
                    for rr in range(16):
                        r = gi * 16 + rr
                        ee = ev_e[rr]
                        eo = ev_o[rr]
                        r0a = (r0a + chunk_v[r, pl.ds(0, 16)] * ee
                               + chunk_v[r, pl.ds(64, 16)] * eo)
                        r1a = (r1a + chunk_v[r, pl.ds(16, 16)] * ee
                               + chunk_v[r, pl.ds(80, 16)] * eo)
                        r2a = (r2a + chunk_v[r, pl.ds(32, 16)] * ee
                               + chunk_v[r, pl.ds(96, 16)] * eo)
                        r3a = (r3a + chunk_v[r, pl.ds(48, 16)] * ee
                               + chunk_v[r, pl.ds(112, 16)] * eo)
                return (m_run, bidx, s_run, r0a, r1a, r2a, r3a)

            init = (neg, jnp.full((16,), K, jnp.int32), zero,
                    zero, zero, zero, zero)
            m_run, bidx, s_run, r0a, r1a, r2a, r3a = lax.fori_loop(
                0, nchunks, chunk_body, init)

            m_fin = jnp.max(m_run)
            cand = jnp.where(m_run >= m_fin, bidx, K)
            idx_fin = jnp.min(cand)
            s_fin = jnp.sum(s_run)

            pidx = idx_fin // 2
            parity = idx_fin - 2 * pidx
            pltpu.sync_copy(bank_hbm.at[b, pl.ds(pidx, 1)], rb_v)
            pltpu.sync_copy(bank_hbm.at[b, pl.ds(0, 1)], r0_v)

            out_v[0, pl.ds(0, 16)] = r0a
            out_v[0, pl.ds(16, 16)] = r1a
            out_v[0, pl.ds(32, 16)] = r2a
            out_v[0, pl.ds(48, 16)] = r3a
            par = zeroi + parity
            for t in range(D // 16):
                he = rb_v[0, pl.ds(16 * t, 16)]
                ho = rb_v[0, pl.ds(D + 16 * t, 16)]
                out_v[1, pl.ds(16 * t, 16)] = jnp.where(par == 0, he, ho)
                out_v[2, pl.ds(16 * t, 16)] = r0_v[0, pl.ds(16 * t, 16)]
            sc0 = (jnp.where(i16 == 0, jnp.full((16,), s_fin), 0.0)
                   + jnp.where(i16 == 3, jnp.full((16,), m_fin), 0.0))
            out_v[3, pl.ds(0, 16)] = sc0
            out_v[3, pl.ds(16, 16)] = zero
            out_v[3, pl.ds(32, 16)] = zero
            out_v[3, pl.ds(48, 16)] = zero
            pltpu.sync_copy(out_v, out_hbm.at[g])
            return 0

        lax.fori_loop(0, spw, stream_body, 0)

    return sc_kernel(bank2, item, query)


def _finalize_body(packed_ref, qr_ref, ir_ref, out_ref):
    pk = packed_ref[...]
    B = pk.shape[0]
    BT = B - _G
    cond = (jnp.sum(pk[:, 3, 3:4]) * (1.0 / B)) >= 0.5

    R_a = pk[0:BT, 0, :]
    S_a = pk[0:BT, 3, 0:1]
    S_fin_a = S_a + jnp.where(cond, pk[0:BT, 3, 1:2], pk[0:BT, 3, 2:3])
    R_fin_a = R_a + jnp.where(cond, pk[0:BT, 1, :], pk[0:BT, 2, :])
    out_ref[0:BT, :] = R_fin_a / S_fin_a

    R_b = pk[BT:, 0, :]
    rb = pk[BT:, 1, :]
    r0 = pk[BT:, 2, :]
    S_b = pk[BT:, 3, 0:1]
    qry = qr_ref[BT:, :]
    itm = ir_ref[BT:, :]

    def inv_norm(x):
        return 1.0 / jnp.maximum(jnp.sqrt(jnp.sum(x * x, axis=1, keepdims=True)), _EPS)

    def rdot(a, bv):
        return jnp.sum(a * bv, axis=1, keepdims=True)

    inv_q = inv_norm(qry)
    inv_i = inv_norm(itm)
    e_best = jnp.exp(rdot(qry, rb) * inv_norm(rb) * inv_q)
    e_0 = jnp.exp(rdot(qry, r0) * inv_norm(r0) * inv_q)
    new_c = 0.5 * rb + 0.5 * itm
    e_new_c = jnp.exp(rdot(qry, new_c) * inv_norm(new_c) * inv_q)
    e_new_o = jnp.exp(rdot(qry, itm) * inv_i * inv_q)

    S_fin_b = S_b + jnp.where(cond, e_new_c - e_best, e_new_o - e_0)
    R_fin_b = R_b + jnp.where(cond,
                              e_new_c * new_c - e_best * rb,
                              e_new_o * itm - e_0 * r0)
    out_ref[BT:, :] = R_fin_b / S_fin_b


def kernel(query, item, memory_bank):
    B, K, D = memory_bank.shape
    BT = B - _G
    q3 = query.reshape(B, 1, D)
    i3 = item.reshape(B, 1, D)
    bank2 = memory_bank.reshape(B, K // 2, 2 * D)

    BPB = 8
    packed_tc = pl.pallas_call(
        _pass_body,
        grid=(BT // BPB,),
        in_specs=[
            pl.BlockSpec((BPB, K // 2, 2 * D), lambda b: (b, 0, 0)),
            pl.BlockSpec((BPB, 1, D), lambda b: (b, 0, 0)),
            pl.BlockSpec((BPB, 1, D), lambda b: (b, 0, 0)),
        ],
        out_specs=pl.BlockSpec((BPB, 4, D), lambda b: (b, 0, 0)),
        out_shape=jax.ShapeDtypeStruct((BT, 4, D), jnp.float32),
    )(bank2, i3, q3)

    packed_sc = _sc_pass(bank2, item, query, B, K, D, BT)
    packed = jnp.concatenate([packed_tc, packed_sc], axis=0)

    retrieved = pl.pallas_call(
        _finalize_body,
        out_shape=jax.ShapeDtypeStruct((B, D), jnp.float32),
    )(packed, query, item)
    return retrieved
